# Initial kernel scaffold; baseline (speedup 1.0000x reference)
import jax
import jax.numpy as jnp
from jax.experimental import pallas as pl


def kernel(y, ln_g, ln_b, rw1, rb1, rw2, rb2, gate_bias, ew1, eb1, ew2, eb2):
    raise NotImplementedError("write your pallas kernel here")


# speedup vs baseline: 3.5322x; 3.5322x over previous
import functools

import jax
import jax.numpy as jnp
from jax.experimental import pallas as pl

B, T, D, H, E = 4, 4096, 36, 256, 8
TOPK = 2
TAU = 1.5
EPS_SMOOTH = 0.02
RES_SCALE = 0.2


def _gelu_exact(x):
    return 0.5 * x * (1.0 + jax.lax.erf(x * 0.7071067811865476))


def _fused_body(y_ref, ln_g_ref, ln_b_ref, rw1_ref, rb1_ref, rw2_ref, rb2_ref,
                gate_bias_ref, w1_ref, b1_ref, w2_ref, eb2_ref, out_ref):
    yb = y_ref[0]

    prev = jnp.concatenate([yb[0:1], yb[:-1]], axis=0)
    trans = jnp.abs(yb - prev).mean(axis=-1, keepdims=True)
    ym2 = jnp.concatenate([yb[0:1], yb[0:1], yb[:-2]], axis=0)
    yp1 = jnp.concatenate([yb[1:], yb[-1:]], axis=0)
    yp2 = jnp.concatenate([yb[2:], yb[-1:], yb[-1:]], axis=0)
    y_ma = (ym2 + prev + yb + yp1 + yp2) * 0.2
    cont = jnp.abs(yb - y_ma).mean(axis=-1, keepdims=True)
    pitch_abs = jnp.abs(jnp.clip(yb[:, 18:19], -2.0, 2.0))
    harm = jnp.clip(yb[:, 19:20], 0.0, 1.0)
    sp = yb[:, 20:36]
    spm = sp.mean(axis=-1, keepdims=True)
    spec_var = ((sp - spm) ** 2).sum(axis=-1, keepdims=True) * (1.0 / 15.0)
    energy = yb[:, 0:1]
    r = jnp.concatenate(
        [trans, cont, harm, spec_var, energy, pitch_abs,
         jnp.zeros((T, 4), yb.dtype)], axis=-1)

    mu = r.mean(axis=-1, keepdims=True)
    var = ((r - mu) ** 2).mean(axis=-1, keepdims=True)
    rn = (r - mu) / jnp.sqrt(var + 1e-5) * ln_g_ref[0] + ln_b_ref[0]

    h = jax.lax.dot_general(rn, rw1_ref[...], (((1,), (1,)), ((), ())),
                            preferred_element_type=jnp.float32) + rb1_ref[0]
    h = _gelu_exact(h)
    logits = jax.lax.dot_general(h, rw2_ref[...], (((1,), (1,)), ((), ())),
                                 preferred_element_type=jnp.float32) + rb2_ref[0]
    logits = logits * (1.0 / TAU) + gate_bias_ref[0]

    z = logits - logits.max(axis=-1, keepdims=True)
    ez = jnp.exp(z)
    p = ez / ez.sum(axis=-1, keepdims=True)
    p = (1.0 - EPS_SMOOTH) * p + EPS_SMOOTH / float(E)

    lane = jax.lax.broadcasted_iota(jnp.int32, (T, E), 1)
    m1 = p.max(axis=-1, keepdims=True)
    idx1 = jnp.where(p == m1, lane, E).min(axis=-1, keepdims=True)
    oh1 = lane == idx1
    p_ex = jnp.where(oh1, -jnp.inf, p)
    m2 = p_ex.max(axis=-1, keepdims=True)
    idx2 = jnp.where(p_ex == m2, lane, E).min(axis=-1, keepdims=True)
    mask = (oh1 | (lane == idx2)).astype(p.dtype)
    probs = p * mask
    probs = probs / (probs.sum(axis=-1, keepdims=True) + 1e-8)

    acc = jax.lax.dot_general(probs, eb2_ref[...], (((1,), (0,)), ((), ())),
                              preferred_element_type=jnp.float32)
    for e in range(E):
        w1e = w1_ref[:, e * H:(e + 1) * H]
        he = jax.lax.dot_general(yb, w1e, (((1,), (0,)), ((), ())),
                                 preferred_element_type=jnp.float32)
        he = _gelu_exact(he + b1_ref[0, e * H:(e + 1) * H])
        he = he * probs[:, e:e + 1]
        w2e = w2_ref[e * H:(e + 1) * H, :]
        acc = acc + jax.lax.dot_general(he, w2e, (((1,), (0,)), ((), ())),
                                        preferred_element_type=jnp.float32)
    out_ref[0] = yb + RES_SCALE * acc


@functools.partial(jax.jit, static_argnames=("interpret",))
def _run(y, ln_g, ln_b, rw1, rb1, rw2, rb2, gate_bias, ew1, eb1, ew2, eb2,
         interpret=False):
    w1 = ew1.transpose(2, 0, 1).reshape(D, E * H)
    b1 = eb1.reshape(1, E * H)
    w2 = ew2.transpose(0, 2, 1).reshape(E * H, D)
    full = lambda shape: pl.BlockSpec(shape, lambda b: (0,) * len(shape))
    return pl.pallas_call(
        _fused_body,
        grid=(B,),
        in_specs=[
            pl.BlockSpec((1, T, D), lambda b: (b, 0, 0)),
            full((1, 10)), full((1, 10)),
            full((16, 10)), full((1, 16)),
            full((E, 16)), full((1, E)),
            full((1, E)),
            full((D, E * H)), full((1, E * H)),
            full((E * H, D)), full((E, D)),
        ],
        out_specs=pl.BlockSpec((1, T, D), lambda b: (b, 0, 0)),
        out_shape=jax.ShapeDtypeStruct((B, T, D), jnp.float32),
        interpret=interpret,
    )(y, ln_g.reshape(1, 10), ln_b.reshape(1, 10), rw1, rb1.reshape(1, 16),
      rw2, rb2.reshape(1, E), gate_bias.reshape(1, E), w1, b1, w2, eb2)


def kernel(y, ln_g, ln_b, rw1, rb1, rw2, rb2, gate_bias, ew1, eb1, ew2, eb2):
    return _run(y, ln_g, ln_b, rw1, rb1, rw2, rb2, gate_bias, ew1, eb1, ew2,
                eb2)


# speedup vs baseline: 3.5859x; 1.0152x over previous
import functools

import jax
import jax.numpy as jnp
from jax.experimental import pallas as pl

B, T, D, H, E = 4, 4096, 36, 256, 8
TOPK = 2
TAU = 1.5
EPS_SMOOTH = 0.02
RES_SCALE = 0.2


def _gelu_exact(x):
    return 0.5 * x * (1.0 + jax.lax.erf(x * 0.7071067811865476))


def _fused_body(y_ref, ln_g_ref, ln_b_ref, rw1_ref, rb1_ref, rw2_ref, rb2_ref,
                gate_bias_ref, w1_ref, b1_ref, w2_ref, eb2_ref, out_ref):
    yb = y_ref[0]

    prev = jnp.concatenate([yb[0:1], yb[:-1]], axis=0)
    trans = jnp.abs(yb - prev).mean(axis=-1, keepdims=True)
    ym2 = jnp.concatenate([yb[0:1], yb[0:1], yb[:-2]], axis=0)
    yp1 = jnp.concatenate([yb[1:], yb[-1:]], axis=0)
    yp2 = jnp.concatenate([yb[2:], yb[-1:], yb[-1:]], axis=0)
    y_ma = (ym2 + prev + yb + yp1 + yp2) * 0.2
    cont = jnp.abs(yb - y_ma).mean(axis=-1, keepdims=True)
    pitch_abs = jnp.abs(jnp.clip(yb[:, 18:19], -2.0, 2.0))
    harm = jnp.clip(yb[:, 19:20], 0.0, 1.0)
    sp = yb[:, 20:36]
    spm = sp.mean(axis=-1, keepdims=True)
    spec_var = ((sp - spm) ** 2).sum(axis=-1, keepdims=True) * (1.0 / 15.0)
    energy = yb[:, 0:1]
    r = jnp.concatenate(
        [trans, cont, harm, spec_var, energy, pitch_abs,
         jnp.zeros((T, 4), yb.dtype)], axis=-1)

    mu = r.mean(axis=-1, keepdims=True)
    var = ((r - mu) ** 2).mean(axis=-1, keepdims=True)
    rn = (r - mu) / jnp.sqrt(var + 1e-5) * ln_g_ref[0] + ln_b_ref[0]

    h = jax.lax.dot_general(rn, rw1_ref[...], (((1,), (1,)), ((), ())),
                            preferred_element_type=jnp.float32) + rb1_ref[0]
    h = _gelu_exact(h)
    logits = jax.lax.dot_general(h, rw2_ref[...], (((1,), (1,)), ((), ())),
                                 preferred_element_type=jnp.float32) + rb2_ref[0]
    logits = logits * (1.0 / TAU) + gate_bias_ref[0]

    z = logits - logits.max(axis=-1, keepdims=True)
    ez = jnp.exp(z)
    p = ez / ez.sum(axis=-1, keepdims=True)
    p = (1.0 - EPS_SMOOTH) * p + EPS_SMOOTH / float(E)

    lane = jax.lax.broadcasted_iota(jnp.int32, (T, E), 1)
    m1 = p.max(axis=-1, keepdims=True)
    idx1 = jnp.where(p == m1, lane, E).min(axis=-1, keepdims=True)
    oh1 = lane == idx1
    p_ex = jnp.where(oh1, -jnp.inf, p)
    m2 = p_ex.max(axis=-1, keepdims=True)
    idx2 = jnp.where(p_ex == m2, lane, E).min(axis=-1, keepdims=True)
    mask = (oh1 | (lane == idx2)).astype(p.dtype)
    probs = p * mask
    probs = probs / (probs.sum(axis=-1, keepdims=True) + 1e-8)

    acc = jax.lax.dot_general(probs, eb2_ref[...], (((1,), (0,)), ((), ())),
                              preferred_element_type=jnp.float32)
    yb16 = yb.astype(jnp.bfloat16)
    for e in range(E):
        w1e = w1_ref[:, e * H:(e + 1) * H]
        he = jax.lax.dot_general(yb16, w1e, (((1,), (0,)), ((), ())),
                                 preferred_element_type=jnp.float32)
        he = _gelu_exact(he + b1_ref[0, e * H:(e + 1) * H])
        w2e = w2_ref[e * H:(e + 1) * H, :]
        oe = jax.lax.dot_general(he.astype(jnp.bfloat16), w2e,
                                 (((1,), (0,)), ((), ())),
                                 preferred_element_type=jnp.float32)
        acc = acc + oe * probs[:, e:e + 1]
    out_ref[0] = yb + RES_SCALE * acc


@functools.partial(jax.jit, static_argnames=("interpret",))
def _run(y, ln_g, ln_b, rw1, rb1, rw2, rb2, gate_bias, ew1, eb1, ew2, eb2,
         interpret=False):
    w1 = ew1.transpose(2, 0, 1).reshape(D, E * H).astype(jnp.bfloat16)
    b1 = eb1.reshape(1, E * H)
    w2 = ew2.transpose(0, 2, 1).reshape(E * H, D).astype(jnp.bfloat16)
    full = lambda shape: pl.BlockSpec(shape, lambda b: (0,) * len(shape))
    return pl.pallas_call(
        _fused_body,
        grid=(B,),
        in_specs=[
            pl.BlockSpec((1, T, D), lambda b: (b, 0, 0)),
            full((1, 10)), full((1, 10)),
            full((16, 10)), full((1, 16)),
            full((E, 16)), full((1, E)),
            full((1, E)),
            full((D, E * H)), full((1, E * H)),
            full((E * H, D)), full((E, D)),
        ],
        out_specs=pl.BlockSpec((1, T, D), lambda b: (b, 0, 0)),
        out_shape=jax.ShapeDtypeStruct((B, T, D), jnp.float32),
        interpret=interpret,
    )(y, ln_g.reshape(1, 10), ln_b.reshape(1, 10), rw1, rb1.reshape(1, 16),
      rw2, rb2.reshape(1, E), gate_bias.reshape(1, E), w1, b1, w2, eb2)


def kernel(y, ln_g, ln_b, rw1, rb1, rw2, rb2, gate_bias, ew1, eb1, ew2, eb2):
    return _run(y, ln_g, ln_b, rw1, rb1, rw2, rb2, gate_bias, ew1, eb1, ew2,
                eb2)


# speedup vs baseline: 4.0485x; 1.1290x over previous
import functools

import jax
import jax.numpy as jnp
from jax.experimental import pallas as pl

B, T, D, H, E = 4, 4096, 36, 256, 8
TOPK = 2
TAU = 1.5
EPS_SMOOTH = 0.02
RES_SCALE = 0.2


def _fused_body(y_ref, g6_ref, svec_ref, cb_ref, rw2t_ref, bvec_ref,
                w1_ref, b1_ref, w2_ref, eb2_ref, c36_ref, c16_ref, out_ref):
    yb = y_ref[...]

    prev = jnp.concatenate([yb[0:1], yb[:-1]], axis=0)
    ym2 = jnp.concatenate([yb[0:1], yb[0:1], yb[:-2]], axis=0)
    yp1 = jnp.concatenate([yb[1:], yb[-1:]], axis=0)
    yp2 = jnp.concatenate([yb[2:], yb[-1:], yb[-1:]], axis=0)
    y_ma = (ym2 + prev + yb + yp1 + yp2) * 0.2
    c36 = c36_ref[...]
    dot = lambda a, b: jax.lax.dot_general(
        a, b, (((1,), (0,)), ((), ())), preferred_element_type=jnp.float32)
    trans = dot(jnp.abs(yb - prev), c36)
    cont = dot(jnp.abs(yb - y_ma), c36)
    pitch_abs = jnp.abs(jnp.clip(yb[:, 18:19], -2.0, 2.0))
    harm = jnp.clip(yb[:, 19:20], 0.0, 1.0)
    sp = yb[:, 20:36]
    c16 = c16_ref[...]
    s1 = dot(sp, c16) * (1.0 / 16.0)
    s2 = dot(sp * sp, c16)
    spec_var = (s2 - 16.0 * s1 * s1) * (1.0 / 15.0)
    energy = yb[:, 0:1]
    r6 = jnp.concatenate([trans, cont, harm, spec_var, energy, pitch_abs],
                         axis=-1)

    ones6 = jnp.ones((6, 1), jnp.float32)
    mu = dot(r6, ones6) * 0.1
    r2s = dot(r6 * r6, ones6) * 0.1
    istd = 1.0 / jnp.sqrt(r2s - mu * mu + 1e-5)
    h_pre = istd * dot(r6, g6_ref[...]) - (mu * istd) * svec_ref[...] \
        + cb_ref[...]
    h = h_pre * (1.0 + jax.lax.erf(h_pre * 0.7071067811865476))

    logits = jax.lax.dot_general(rw2t_ref[...], h, (((1,), (1,)), ((), ())),
                                 preferred_element_type=jnp.float32) \
        + bvec_ref[...]

    z = logits - logits.max(axis=0, keepdims=True)
    ez = jnp.exp(z)
    p = ez / ez.sum(axis=0, keepdims=True)
    p = (1.0 - EPS_SMOOTH) * p + EPS_SMOOTH / float(E)
    srow = jax.lax.broadcasted_iota(jnp.int32, (E, T), 0)
    m1 = p.max(axis=0, keepdims=True)
    idx1 = jnp.where(p == m1, srow, E).min(axis=0, keepdims=True)
    oh1 = srow == idx1
    p_ex = jnp.where(oh1, -jnp.inf, p)
    m2 = p_ex.max(axis=0, keepdims=True)
    idx2 = jnp.where(p_ex == m2, srow, E).min(axis=0, keepdims=True)
    pm = p * (oh1 | (srow == idx2)).astype(p.dtype)
    probs_t = pm / (pm.sum(axis=0, keepdims=True) + 1e-8)
    probs = jnp.transpose(probs_t, (1, 0))

    acc = dot(probs, eb2_ref[...])
    yb16 = yb.astype(jnp.bfloat16)
    for e in range(E):
        he = jax.lax.dot_general(yb16, w1_ref[:, e * H:(e + 1) * H],
                                 (((1,), (0,)), ((), ())),
                                 preferred_element_type=jnp.float32)
        he = he + b1_ref[0, e * H:(e + 1) * H]
        he = he * (1.0 + jax.lax.erf(he * 0.7071067811865476))
        oe = jax.lax.dot_general(he.astype(jnp.bfloat16),
                                 w2_ref[e * H:(e + 1) * H, :],
                                 (((1,), (0,)), ((), ())),
                                 preferred_element_type=jnp.float32)
        acc = acc + oe * probs[:, e:e + 1]
    out_ref[...] = yb + RES_SCALE * acc


@functools.partial(jax.jit, static_argnames=("interpret",))
def _run(y, ln_g, ln_b, rw1, rb1, rw2, rb2, gate_bias, ew1, eb1, ew2, eb2,
         interpret=False):
    w1 = ew1.transpose(2, 0, 1).reshape(D, E * H).astype(jnp.bfloat16)
    b1 = eb1.reshape(1, E * H)
    w2 = (0.5 * ew2.transpose(0, 2, 1).reshape(E * H, D)).astype(jnp.bfloat16)
    g_rw = ln_g[:, None] * rw1.T
    g6 = g_rw[:6]
    svec = jnp.sum(g_rw, axis=0, keepdims=True)
    cb = (ln_b @ rw1.T + rb1)[None]
    rw2t = rw2 * (0.5 / TAU)
    bvec = (rb2 / TAU + gate_bias)[:, None]
    c36 = jnp.full((D, 1), 1.0 / 36.0, jnp.float32)
    c16 = jnp.ones((16, 1), jnp.float32)
    full = lambda shape: pl.BlockSpec(shape, lambda b: (0,) * len(shape))
    out = pl.pallas_call(
        _fused_body,
        grid=(B,),
        in_specs=[
            pl.BlockSpec((T, D), lambda b: (b, 0)),
            full((6, 16)), full((1, 16)), full((1, 16)),
            full((E, 16)), full((E, 1)),
            full((D, E * H)), full((1, E * H)),
            full((E * H, D)), full((E, D)),
            full((D, 1)), full((16, 1)),
        ],
        out_specs=pl.BlockSpec((T, D), lambda b: (b, 0)),
        out_shape=jax.ShapeDtypeStruct((B * T, D), jnp.float32),
        interpret=interpret,
    )(y.reshape(B * T, D), g6, svec, cb, rw2t, bvec, w1, b1, w2, eb2,
      c36, c16)
    return out.reshape(B, T, D)


def kernel(y, ln_g, ln_b, rw1, rb1, rw2, rb2, gate_bias, ew1, eb1, ew2, eb2):
    return _run(y, ln_g, ln_b, rw1, rb1, rw2, rb2, gate_bias, ew1, eb1, ew2,
                eb2)
